# baseline (speedup 1.0000x reference)
import functools

import jax
import jax.numpy as jnp
from jax import lax
from jax.experimental import pallas as pl
from jax.experimental.pallas import tpu as pltpu
from jax.experimental.pallas import tpu_sc as plsc

_NW = 32
_NB = 2


def _gather_kernel(batches_per_w, hist, idx_hbm, table_hbm, out_hbm,
                   idx_v, rows0, rows1, sg0, sg1, so0, so1):
    wid = lax.axis_index("s") * 2 + lax.axis_index("c")
    b_base = wid * batches_per_w
    pltpu.sync_copy(idx_hbm.at[pl.ds(b_base, batches_per_w)], idx_v)

    splits = [(0, 128), (128, hist - 128)] if hist > 128 else [(0, hist)]

    def fire_group(g, rows, sem):
        for b in range(_NB):
            r = g * _NB + b
            for (off, ln) in splits:
                pltpu.async_copy(
                    table_hbm.at[idx_v.at[r, pl.ds(off, ln)]],
                    rows.at[b, pl.ds(off, ln)],
                    sem)

    def fire_out(g, rows, sem):
        pltpu.async_copy(
            rows, out_hbm.at[pl.ds(b_base + g * _NB, _NB)], sem)

    def drain_gather(rows, sem):
        pltpu.make_async_copy(
            out_hbm.at[pl.ds(b_base, _NB)], rows, sem).wait()

    def drain_out(rows, sem):
        pltpu.make_async_copy(
            rows, out_hbm.at[pl.ds(b_base, _NB)], sem).wait()

    fire_group(0, rows0, sg0)
    npairs = batches_per_w // _NB // 2

    def body(t, _):
        a = 2 * t

        @pl.when(t > 0)
        def _():
            drain_out(rows1, so1)

        fire_group(a + 1, rows1, sg1)
        drain_gather(rows0, sg0)
        fire_out(a, rows0, so0)
        drain_out(rows0, so0)

        @pl.when(t < npairs - 1)
        def _():
            fire_group(a + 2, rows0, sg0)

        drain_gather(rows1, sg1)
        fire_out(a + 1, rows1, so1)
        return 0

    lax.fori_loop(0, npairs, body, 0)
    drain_out(rows1, so1)


def kernel(paragraph_variable, embedding_table):
    B, H = paragraph_variable.shape
    V, D = embedding_table.shape
    batches_per_w = B // _NW

    mesh = plsc.VectorSubcoreMesh(core_axis_name="c", subcore_axis_name="s")
    run = pl.kernel(
        functools.partial(_gather_kernel, batches_per_w, H),
        mesh=mesh,
        out_type=jax.ShapeDtypeStruct((B, H, D), jnp.float32),
        scratch_types=[
            pltpu.VMEM((batches_per_w, H), jnp.int32),
            pltpu.VMEM((_NB, H, D), jnp.float32),
            pltpu.VMEM((_NB, H, D), jnp.float32),
            pltpu.SemaphoreType.DMA,
            pltpu.SemaphoreType.DMA,
            pltpu.SemaphoreType.DMA,
            pltpu.SemaphoreType.DMA,
        ],
        compiler_params=pltpu.CompilerParams(use_tc_tiling_on_sc=False),
    )
    return run(paragraph_variable, embedding_table)


# speedup vs baseline: 1.2207x; 1.2207x over previous
import functools

import jax
import jax.numpy as jnp
from jax import lax
from jax.experimental import pallas as pl
from jax.experimental.pallas import tpu as pltpu
from jax.experimental.pallas import tpu_sc as plsc

_NW = 32


def _gather_kernel(batches_per_w, hist, idx_hbm, wide_hbm, out_hbm,
                   idx_v, rows0, rows1, sg0, sg1, so0, so1):
    wid = lax.axis_index("s") * 2 + lax.axis_index("c")
    b_base = wid * batches_per_w
    pltpu.sync_copy(idx_hbm.at[pl.ds(b_base, batches_per_w)], idx_v)

    splits = [(0, 128), (128, hist - 128)] if hist > 128 else [(0, hist)]

    def fire_group(g, rows, sem):
        for (off, ln) in splits:
            pltpu.async_copy(
                wide_hbm.at[idx_v.at[g, pl.ds(off, ln)]],
                rows.at[pl.ds(off, ln)],
                sem)

    def fire_out(g, rows, sem):
        pltpu.async_copy(rows, out_hbm.at[b_base + g], sem)

    def drain_gather(rows, sem):
        pltpu.make_async_copy(
            wide_hbm.at[pl.ds(0, hist)], rows, sem).wait()

    def drain_out(rows, sem):
        pltpu.make_async_copy(rows, out_hbm.at[b_base], sem).wait()

    fire_group(0, rows0, sg0)
    npairs = batches_per_w // 2

    def body(t, _):
        a = 2 * t

        @pl.when(t > 0)
        def _():
            drain_out(rows1, so1)

        fire_group(a + 1, rows1, sg1)
        drain_gather(rows0, sg0)
        fire_out(a, rows0, so0)
        drain_out(rows0, so0)

        @pl.when(t < npairs - 1)
        def _():
            fire_group(a + 2, rows0, sg0)

        drain_gather(rows1, sg1)
        fire_out(a + 1, rows1, so1)
        return 0

    lax.fori_loop(0, npairs, body, 0)
    drain_out(rows1, so1)


def kernel(paragraph_variable, embedding_table):
    B, H = paragraph_variable.shape
    V, D = embedding_table.shape
    batches_per_w = B // _NW

    wide = jnp.pad(embedding_table, ((0, 0), (0, 2 * D - 64)))

    mesh = plsc.VectorSubcoreMesh(core_axis_name="c", subcore_axis_name="s")
    gather = pl.kernel(
        functools.partial(_gather_kernel, batches_per_w, H),
        mesh=mesh,
        out_type=jax.ShapeDtypeStruct((B, H, 2 * D), jnp.float32),
        scratch_types=[
            pltpu.VMEM((batches_per_w, H), jnp.int32),
            pltpu.VMEM((H, 2 * D), jnp.float32),
            pltpu.VMEM((H, 2 * D), jnp.float32),
            pltpu.SemaphoreType.DMA,
            pltpu.SemaphoreType.DMA,
            pltpu.SemaphoreType.DMA,
            pltpu.SemaphoreType.DMA,
        ],
    )
    res = gather(paragraph_variable, wide)
    return res[:, :, :D]
